# baseline (speedup 1.0000x reference)
import functools

import jax
import jax.numpy as jnp
from jax import lax
from jax.experimental import pallas as pl
from jax.experimental.pallas import tpu as pltpu
from jax.experimental.pallas import tpu_sc as plsc

_N = 10000
_E = 320000
_H = 128
_NC = 2
_NS = 16
_NW = _NC * _NS
_NP = 10240
_EP = 327680
_ER = _EP // 128
_RW = _ER // _NW
_CR = 2
_CI = _RW // _CR
_RPS = _NP // _NS
_BR = 1024

_mesh = plsc.VectorSubcoreMesh(
    core_axis_name="c", subcore_axis_name="s", num_cores=_NC, num_subcores=_NS
)


_DW = 128


@functools.partial(
    pl.kernel,
    out_type=jax.ShapeDtypeStruct((_NC, _NP, _DW), jnp.float32),
    mesh=_mesh,
    scratch_types=[
        pltpu.VMEM((_RW, 128), jnp.int32),
        pltpu.VMEM((128, _DW), jnp.float32),
        pltpu.VMEM((64, _DW), jnp.float32),
        pltpu.VMEM_SHARED((_NP, _DW), jnp.float32),
        pltpu.SemaphoreType.DMA,
    ],
)
def _sc_deg(dst_hbm, out_hbm, dst_v, ones_v, zv, acc, semi):
    c = lax.axis_index("c")
    s = lax.axis_index("s")
    w = s * _NC + c
    cp_idx = pltpu.async_copy(dst_hbm.at[pl.ds(w * _RW, _RW)], dst_v, semi)

    def initrow(t, carry):
        for j in range(_DW // 16):
            ones_v[t, pl.ds(j * 16, 16)] = jnp.ones((16,), jnp.float32)
            zv[t % 64, pl.ds(j * 16, 16)] = jnp.zeros((16,), jnp.float32)
        return carry

    lax.fori_loop(0, 128, initrow, 0)
    for m in range(_RPS // 64):
        pltpu.sync_copy(zv.at[pl.ds(0, 64)], acc.at[pl.ds(s * _RPS + m * 64, 64)])
    cp_idx.wait()
    plsc.subcore_barrier()

    def body(j, carry):
        pltpu.sync_copy(ones_v, acc.at[dst_v.at[j]], add=True)
        return carry

    lax.fori_loop(0, _RW, body, 0)
    plsc.subcore_barrier()
    pltpu.sync_copy(acc.at[pl.ds(s * _RPS, _RPS)], out_hbm.at[c, pl.ds(s * _RPS, _RPS)])


@functools.partial(
    pl.kernel,
    out_type=jax.ShapeDtypeStruct((_NC, _NP, _H), jnp.float32),
    mesh=_mesh,
    scratch_types=[
        pltpu.VMEM((_RW // 5, 128), jnp.int32),
        pltpu.VMEM((_RW // 5, 128), jnp.int32),
        pltpu.VMEM((_RW // 5, 128), jnp.int32),
        pltpu.VMEM((_RW // 5, 128), jnp.int32),
        pltpu.VMEM((128, _H), jnp.float32),
        pltpu.VMEM((128, _H), jnp.float32),
        pltpu.VMEM((16, _H), jnp.float32),
        pltpu.VMEM_SHARED((_NP, _H), jnp.float32),
        pltpu.SemaphoreType.DMA,
        pltpu.SemaphoreType.DMA,
        pltpu.SemaphoreType.DMA,
        pltpu.SemaphoreType.DMA,
    ],
)
def _sc_agg(y_hbm, src_hbm, dst_hbm, out_hbm, src_a, dst_a, src_b, dst_b,
            buf0, buf1, zb, acc, sem0, sem1, semia, semib):
    c = lax.axis_index("c")
    s = lax.axis_index("s")
    w = s * _NC + c
    r0 = w * _RW
    _PR = _RW // 5
    sets = ((src_a, dst_a, semia), (src_b, dst_b, semib))
    pltpu.async_copy(src_hbm.at[pl.ds(r0, _PR)], src_a, semia)
    pltpu.async_copy(dst_hbm.at[pl.ds(r0, _PR)], dst_a, semia)
    for i in range(16):
        for j in range(_H // 16):
            zb[i, pl.ds(j * 16, 16)] = jnp.zeros((16,), jnp.float32)
    for m in range(_RPS // 16):
        pltpu.sync_copy(zb, acc.at[pl.ds(s * _RPS + m * 16, 16)])
    plsc.subcore_barrier()

    bufs = (buf0, buf1)
    sems = (sem0, sem1)
    for ph in range(5):
        src_v, dst_v, semi = sets[ph % 2]
        rp = r0 + ph * _PR
        pltpu.make_async_copy(src_hbm.at[pl.ds(rp, _PR)], src_v, semi).wait()
        pltpu.make_async_copy(dst_hbm.at[pl.ds(rp, _PR)], dst_v, semi).wait()
        if ph + 1 < 5:
            nsrc, ndst, nsem = sets[(ph + 1) % 2]
            rn = r0 + (ph + 1) * _PR
            pltpu.async_copy(src_hbm.at[pl.ds(rn, _PR)], nsrc, nsem)
            pltpu.async_copy(dst_hbm.at[pl.ds(rn, _PR)], ndst, nsem)
        for b in range(2):
            pltpu.async_copy(y_hbm.at[src_v.at[b]], bufs[b], sems[b])

        def body(k, carry):
            j0 = 2 * k
            for b in range(2):
                j = j0 + b
                pltpu.make_async_copy(y_hbm.at[src_v.at[j]], bufs[b], sems[b]).wait()
                pltpu.sync_copy(bufs[b], acc.at[dst_v.at[j]], add=True)
                pltpu.async_copy(y_hbm.at[src_v.at[j + 2]], bufs[b], sems[b])
            return carry

        lax.fori_loop(0, (_PR - 2) // 2, body, 0)
        for b in range(2):
            j = _PR - 2 + b
            pltpu.make_async_copy(y_hbm.at[src_v.at[j]], bufs[b], sems[b]).wait()
            pltpu.sync_copy(bufs[b], acc.at[dst_v.at[j]], add=True)
    plsc.subcore_barrier()
    pltpu.sync_copy(
        acc.at[pl.ds(s * _RPS, _RPS)], out_hbm.at[c, pl.ds(s * _RPS, _RPS)]
    )


def _tc0_body(degp_ref, x_ref, ew_ref, eb_ref, w0_ref, dinv_ref, y_ref):
    deg = degp_ref[0, :, 0:1] + degp_ref[1, :, 0:1] + 1.0
    dinv = lax.rsqrt(deg)
    dinv_ref[...] = dinv
    h = jnp.dot(x_ref[...], ew_ref[...], preferred_element_type=jnp.float32)
    h = h + eb_ref[...]
    y_ref[...] = jnp.dot(h, w0_ref[...], preferred_element_type=jnp.float32) * dinv


_tc0 = pl.pallas_call(
    _tc0_body,
    grid=(_NP // _BR,),
    in_specs=[
        pl.BlockSpec((2, _BR, _DW), lambda i: (0, i, 0)),
        pl.BlockSpec((_BR, _H), lambda i: (i, 0)),
        pl.BlockSpec((_H, _H), lambda i: (0, 0)),
        pl.BlockSpec((1, _H), lambda i: (0, 0)),
        pl.BlockSpec((_H, _H), lambda i: (0, 0)),
    ],
    out_specs=[
        pl.BlockSpec((_BR, 1), lambda i: (i, 0)),
        pl.BlockSpec((_BR, _H), lambda i: (i, 0)),
    ],
    out_shape=[
        jax.ShapeDtypeStruct((_NP, 1), jnp.float32),
        jax.ShapeDtypeStruct((_NP, _H), jnp.float32),
    ],
)


def _tc_mid_body(p_ref, yp_ref, dinv_ref, b_ref, w_ref, y_ref):
    dinv = dinv_ref[...]
    h = dinv * (p_ref[0] + p_ref[1] + yp_ref[...]) + b_ref[...]
    y_ref[...] = jnp.dot(h, w_ref[...], preferred_element_type=jnp.float32) * dinv


_tc_mid = pl.pallas_call(
    _tc_mid_body,
    grid=(_NP // _BR,),
    in_specs=[
        pl.BlockSpec((2, _BR, _H), lambda i: (0, i, 0)),
        pl.BlockSpec((_BR, _H), lambda i: (i, 0)),
        pl.BlockSpec((_BR, 1), lambda i: (i, 0)),
        pl.BlockSpec((1, _H), lambda i: (0, 0)),
        pl.BlockSpec((_H, _H), lambda i: (0, 0)),
    ],
    out_specs=pl.BlockSpec((_BR, _H), lambda i: (i, 0)),
    out_shape=jax.ShapeDtypeStruct((_NP, _H), jnp.float32),
)


def _tc_fin_body(p_ref, yp_ref, dinv_ref, b_ref, w_ref, db_ref, o_ref):
    h = dinv_ref[...] * (p_ref[0] + p_ref[1] + yp_ref[...]) + b_ref[...]
    o_ref[...] = jnp.dot(h, w_ref[...], preferred_element_type=jnp.float32) + db_ref[...]


_tc_fin = pl.pallas_call(
    _tc_fin_body,
    grid=(_NP // _BR,),
    in_specs=[
        pl.BlockSpec((2, _BR, _H), lambda i: (0, i, 0)),
        pl.BlockSpec((_BR, _H), lambda i: (i, 0)),
        pl.BlockSpec((_BR, 1), lambda i: (i, 0)),
        pl.BlockSpec((1, _H), lambda i: (0, 0)),
        pl.BlockSpec((_H, _H), lambda i: (0, 0)),
        pl.BlockSpec((1, _H), lambda i: (0, 0)),
    ],
    out_specs=pl.BlockSpec((_BR, _H), lambda i: (i, 0)),
    out_shape=jax.ShapeDtypeStruct((_N, _H), jnp.float32),
)


def kernel(x, edge_index, emb_W, emb_b, W0, b0, W1, b1, W2, b2, dec_W, dec_b):
    src = edge_index[0]
    dst = edge_index[1]
    pad = jnp.full((_EP - _E,), _N, dtype=edge_index.dtype)
    src2 = jnp.concatenate([src, pad]).reshape(_ER, 128)
    dst2 = jnp.concatenate([dst, pad]).reshape(_ER, 128)
    x_p = jnp.concatenate([x, jnp.zeros((_NP - _N, x.shape[1]), x.dtype)])

    degp = _sc_deg(dst2)
    dinv, y = _tc0(degp, x_p, emb_W, emb_b.reshape(1, _H), W0)
    p = _sc_agg(y, src2, dst2)
    y = _tc_mid(p, y, dinv, b0.reshape(1, _H), W1)
    p = _sc_agg(y, src2, dst2)
    y = _tc_mid(p, y, dinv, b1.reshape(1, _H), W2)
    p = _sc_agg(y, src2, dst2)
    out = _tc_fin(p, y, dinv, b2.reshape(1, _H), dec_W, dec_b.reshape(1, _H))
    return out


# speedup vs baseline: 3.1178x; 3.1178x over previous
import functools

import jax
import jax.numpy as jnp
from jax import lax
from jax.experimental import pallas as pl
from jax.experimental.pallas import tpu as pltpu
from jax.experimental.pallas import tpu_sc as plsc

_N = 10000
_E = 320000
_H = 128
_NC = 2
_NS = 16
_NW = _NC * _NS
_NP = 10240
_EP = 327680
_ER = _EP // 128
_RW = _ER // _NW
_CR = 2
_CI = _RW // _CR
_RPS = _NP // _NS
_BR = 1024

_mesh = plsc.VectorSubcoreMesh(
    core_axis_name="c", subcore_axis_name="s", num_cores=_NC, num_subcores=_NS
)


_DW = 128


@functools.partial(
    pl.kernel,
    out_type=jax.ShapeDtypeStruct((_NC, _NP, _DW), jnp.float32),
    mesh=_mesh,
    scratch_types=[
        pltpu.VMEM((_RW, 128), jnp.int32),
        pltpu.VMEM((128, _DW), jnp.float32),
        pltpu.VMEM((64, _DW), jnp.float32),
        pltpu.VMEM_SHARED((_NP, _DW), jnp.float32),
        pltpu.SemaphoreType.DMA,
    ],
)
def _sc_deg(dst_hbm, out_hbm, dst_v, ones_v, zv, acc, semi):
    c = lax.axis_index("c")
    s = lax.axis_index("s")
    w = s * _NC + c
    cp_idx = pltpu.async_copy(dst_hbm.at[pl.ds(w * _RW, _RW)], dst_v, semi)

    def initrow(t, carry):
        for j in range(_DW // 16):
            ones_v[t, pl.ds(j * 16, 16)] = jnp.ones((16,), jnp.float32)
            zv[t % 64, pl.ds(j * 16, 16)] = jnp.zeros((16,), jnp.float32)
        return carry

    lax.fori_loop(0, 128, initrow, 0)
    for m in range(_RPS // 64):
        pltpu.sync_copy(zv.at[pl.ds(0, 64)], acc.at[pl.ds(s * _RPS + m * 64, 64)])
    cp_idx.wait()
    plsc.subcore_barrier()

    def body(j, carry):
        pltpu.sync_copy(ones_v, acc.at[dst_v.at[j]], add=True)
        return carry

    lax.fori_loop(0, _RW, body, 0)
    plsc.subcore_barrier()
    pltpu.sync_copy(acc.at[pl.ds(s * _RPS, _RPS)], out_hbm.at[c, pl.ds(s * _RPS, _RPS)])


@functools.partial(
    pl.kernel,
    out_type=jax.ShapeDtypeStruct((_NC, _NP, _H), jnp.float32),
    mesh=_mesh,
    scratch_types=[
        pltpu.VMEM((_RW // 5, 128), jnp.int32),
        pltpu.VMEM((_RW // 5, 128), jnp.int32),
        pltpu.VMEM((_RW // 5, 128), jnp.int32),
        pltpu.VMEM((_RW // 5, 128), jnp.int32),
        pltpu.VMEM((128, _H), jnp.float32),
        pltpu.VMEM((128, _H), jnp.float32),
        pltpu.VMEM((16, _H), jnp.float32),
        pltpu.VMEM_SHARED((_NP, _H), jnp.float32),
        pltpu.SemaphoreType.DMA,
        pltpu.SemaphoreType.DMA,
        pltpu.SemaphoreType.DMA,
        pltpu.SemaphoreType.DMA,
    ],
)
def _sc_agg(y_hbm, src_hbm, dst_hbm, out_hbm, src_a, dst_a, src_b, dst_b,
            buf0, buf1, zb, acc, sem0, sem1, semia, semib):
    c = lax.axis_index("c")
    s = lax.axis_index("s")
    w = s * _NC + c
    r0 = w * _RW
    _PR = _RW // 5
    sets = ((src_a, dst_a, semia), (src_b, dst_b, semib))
    pltpu.async_copy(src_hbm.at[pl.ds(r0, _PR)], src_a, semia)
    pltpu.async_copy(dst_hbm.at[pl.ds(r0, _PR)], dst_a, semia)
    for i in range(16):
        for j in range(_H // 16):
            zb[i, pl.ds(j * 16, 16)] = jnp.zeros((16,), jnp.float32)
    for m in range(_RPS // 16):
        pltpu.sync_copy(zb, acc.at[pl.ds(s * _RPS + m * 16, 16)])
    plsc.subcore_barrier()

    bufs = (buf0, buf1)
    sems = (sem0, sem1)
    for ph in range(5):
        src_v, dst_v, semi = sets[ph % 2]
        rp = r0 + ph * _PR
        pltpu.make_async_copy(src_hbm.at[pl.ds(rp, _PR)], src_v, semi).wait()
        pltpu.make_async_copy(dst_hbm.at[pl.ds(rp, _PR)], dst_v, semi).wait()
        if ph + 1 < 5:
            nsrc, ndst, nsem = sets[(ph + 1) % 2]
            rn = r0 + (ph + 1) * _PR
            pltpu.async_copy(src_hbm.at[pl.ds(rn, _PR)], nsrc, nsem)
            pltpu.async_copy(dst_hbm.at[pl.ds(rn, _PR)], ndst, nsem)
        for b in range(2):
            pltpu.async_copy(y_hbm.at[src_v.at[b]], bufs[b], sems[b])

        def body(k, carry):
            j0 = 2 * k
            for b in range(2):
                j = j0 + b
                pltpu.make_async_copy(y_hbm.at[src_v.at[j]], bufs[b], sems[b]).wait()
                pltpu.sync_copy(bufs[b], acc.at[dst_v.at[j]], add=True)
                pltpu.async_copy(y_hbm.at[src_v.at[j + 2]], bufs[b], sems[b])
            return carry

        lax.fori_loop(0, (_PR - 2) // 2, body, 0)
        for b in range(2):
            j = _PR - 2 + b
            pltpu.make_async_copy(y_hbm.at[src_v.at[j]], bufs[b], sems[b]).wait()
            pltpu.sync_copy(bufs[b], acc.at[dst_v.at[j]], add=True)
    plsc.subcore_barrier()
    pltpu.sync_copy(
        acc.at[pl.ds(s * _RPS, _RPS)], out_hbm.at[c, pl.ds(s * _RPS, _RPS)]
    )


def _tc0_body(degp_ref, x_ref, ew_ref, eb_ref, w0_ref, dinv_ref, y_ref):
    deg = degp_ref[0, :, 0:1] + degp_ref[1, :, 0:1] + 1.0
    dinv = lax.rsqrt(deg)
    dinv_ref[...] = dinv
    h = jnp.dot(x_ref[...], ew_ref[...], preferred_element_type=jnp.float32)
    h = h + eb_ref[...]
    y_ref[...] = jnp.dot(h, w0_ref[...], preferred_element_type=jnp.float32) * dinv


_tc0 = pl.pallas_call(
    _tc0_body,
    grid=(_NP // _BR,),
    in_specs=[
        pl.BlockSpec((2, _BR, _DW), lambda i: (0, i, 0)),
        pl.BlockSpec((_BR, _H), lambda i: (i, 0)),
        pl.BlockSpec((_H, _H), lambda i: (0, 0)),
        pl.BlockSpec((1, _H), lambda i: (0, 0)),
        pl.BlockSpec((_H, _H), lambda i: (0, 0)),
    ],
    out_specs=[
        pl.BlockSpec((_BR, 1), lambda i: (i, 0)),
        pl.BlockSpec((_BR, _H), lambda i: (i, 0)),
    ],
    out_shape=[
        jax.ShapeDtypeStruct((_NP, 1), jnp.float32),
        jax.ShapeDtypeStruct((_NP, _H), jnp.float32),
    ],
)


def _tc_mid_body(p_ref, yp_ref, dinv_ref, b_ref, w_ref, y_ref):
    dinv = dinv_ref[...]
    h = dinv * (p_ref[0] + p_ref[1] + yp_ref[...]) + b_ref[...]
    y_ref[...] = jnp.dot(h, w_ref[...], preferred_element_type=jnp.float32) * dinv


_tc_mid = pl.pallas_call(
    _tc_mid_body,
    grid=(_NP // _BR,),
    in_specs=[
        pl.BlockSpec((2, _BR, _H), lambda i: (0, i, 0)),
        pl.BlockSpec((_BR, _H), lambda i: (i, 0)),
        pl.BlockSpec((_BR, 1), lambda i: (i, 0)),
        pl.BlockSpec((1, _H), lambda i: (0, 0)),
        pl.BlockSpec((_H, _H), lambda i: (0, 0)),
    ],
    out_specs=pl.BlockSpec((_BR, _H), lambda i: (i, 0)),
    out_shape=jax.ShapeDtypeStruct((_NP, _H), jnp.float32),
)


def _tc_fin_body(p_ref, yp_ref, dinv_ref, b_ref, w_ref, db_ref, o_ref):
    h = dinv_ref[...] * (p_ref[0] + p_ref[1] + yp_ref[...]) + b_ref[...]
    o_ref[...] = jnp.dot(h, w_ref[...], preferred_element_type=jnp.float32) + db_ref[...]


_tc_fin = pl.pallas_call(
    _tc_fin_body,
    grid=(_NP // _BR,),
    in_specs=[
        pl.BlockSpec((2, _BR, _H), lambda i: (0, i, 0)),
        pl.BlockSpec((_BR, _H), lambda i: (i, 0)),
        pl.BlockSpec((_BR, 1), lambda i: (i, 0)),
        pl.BlockSpec((1, _H), lambda i: (0, 0)),
        pl.BlockSpec((_H, _H), lambda i: (0, 0)),
        pl.BlockSpec((1, _H), lambda i: (0, 0)),
    ],
    out_specs=pl.BlockSpec((_BR, _H), lambda i: (i, 0)),
    out_shape=jax.ShapeDtypeStruct((_N, _H), jnp.float32),
)


def kernel(x, edge_index, emb_W, emb_b, W0, b0, W1, b1, W2, b2, dec_W, dec_b):
    src = edge_index[0]
    dst = edge_index[1]
    pad = _N + (jnp.arange(_EP - _E, dtype=edge_index.dtype) % (_NP - _N))
    src2 = jnp.concatenate([src, pad]).reshape(_ER, 128)
    dst2 = jnp.concatenate([dst, pad]).reshape(_ER, 128)
    x_p = jnp.concatenate([x, jnp.zeros((_NP - _N, x.shape[1]), x.dtype)])

    degp = _sc_deg(dst2)
    dinv, y = _tc0(degp, x_p, emb_W, emb_b.reshape(1, _H), W0)
    p = _sc_agg(y, src2, dst2)
    y = _tc_mid(p, y, dinv, b0.reshape(1, _H), W1)
    p = _sc_agg(y, src2, dst2)
    y = _tc_mid(p, y, dinv, b1.reshape(1, _H), W2)
    p = _sc_agg(y, src2, dst2)
    out = _tc_fin(p, y, dinv, b2.reshape(1, _H), dec_W, dec_b.reshape(1, _H))
    return out


# speedup vs baseline: 3.1666x; 1.0156x over previous
import functools

import jax
import jax.numpy as jnp
from jax import lax
from jax.experimental import pallas as pl
from jax.experimental.pallas import tpu as pltpu
from jax.experimental.pallas import tpu_sc as plsc

_N = 10000
_E = 320000
_H = 128
_NC = 2
_NS = 16
_NW = _NC * _NS
_NP = 10240
_EP = 327680
_ER = _EP // 128
_RW = _ER // _NW
_CR = 2
_CI = _RW // _CR
_RPS = _NP // _NS
_BR = 1024

_mesh = plsc.VectorSubcoreMesh(
    core_axis_name="c", subcore_axis_name="s", num_cores=_NC, num_subcores=_NS
)


_DW = 128


@functools.partial(
    pl.kernel,
    out_type=jax.ShapeDtypeStruct((_NC, _NP, _DW), jnp.float32),
    mesh=_mesh,
    scratch_types=[
        pltpu.VMEM((_RW, 128), jnp.int32),
        pltpu.VMEM((128, _DW), jnp.float32),
        pltpu.VMEM((64, _DW), jnp.float32),
        pltpu.VMEM_SHARED((_NP, _DW), jnp.float32),
        pltpu.SemaphoreType.DMA,
    ],
)
def _sc_deg(dst_hbm, out_hbm, dst_v, ones_v, zv, acc, semi):
    c = lax.axis_index("c")
    s = lax.axis_index("s")
    w = s * _NC + c
    cp_idx = pltpu.async_copy(dst_hbm.at[pl.ds(w * _RW, _RW)], dst_v, semi)

    def initrow(t, carry):
        for j in range(_DW // 16):
            ones_v[t, pl.ds(j * 16, 16)] = jnp.ones((16,), jnp.float32)
            zv[t % 64, pl.ds(j * 16, 16)] = jnp.zeros((16,), jnp.float32)
        return carry

    lax.fori_loop(0, 128, initrow, 0)
    zcps = [
        pltpu.async_copy(zv.at[pl.ds(0, 64)], acc.at[pl.ds(s * _RPS + m * 64, 64)], semi)
        for m in range(_RPS // 64)
    ]
    for cp in zcps:
        cp.wait()
    cp_idx.wait()
    plsc.subcore_barrier()

    def body(j, carry):
        pltpu.sync_copy(ones_v, acc.at[dst_v.at[j]], add=True)
        return carry

    lax.fori_loop(0, _RW, body, 0)
    plsc.subcore_barrier()
    pltpu.sync_copy(acc.at[pl.ds(s * _RPS, _RPS)], out_hbm.at[c, pl.ds(s * _RPS, _RPS)])


@functools.partial(
    pl.kernel,
    out_type=jax.ShapeDtypeStruct((_NC, _NP, _H), jnp.float32),
    mesh=_mesh,
    scratch_types=[
        pltpu.VMEM((_RW // 5, 128), jnp.int32),
        pltpu.VMEM((_RW // 5, 128), jnp.int32),
        pltpu.VMEM((_RW // 5, 128), jnp.int32),
        pltpu.VMEM((_RW // 5, 128), jnp.int32),
        pltpu.VMEM((128, _H), jnp.float32),
        pltpu.VMEM((128, _H), jnp.float32),
        pltpu.VMEM((16, _H), jnp.float32),
        pltpu.VMEM_SHARED((_NP, _H), jnp.float32),
        pltpu.SemaphoreType.DMA,
        pltpu.SemaphoreType.DMA,
        pltpu.SemaphoreType.DMA,
        pltpu.SemaphoreType.DMA,
        pltpu.SemaphoreType.DMA,
    ],
)
def _sc_agg(y_hbm, src_hbm, dst_hbm, out_hbm, src_a, dst_a, src_b, dst_b,
            buf0, buf1, zb, acc, sem0, sem1, semia, semib, semz):
    c = lax.axis_index("c")
    s = lax.axis_index("s")
    w = s * _NC + c
    r0 = w * _RW
    _PR = _RW // 5
    sets = ((src_a, dst_a, semia), (src_b, dst_b, semib))
    pltpu.async_copy(src_hbm.at[pl.ds(r0, _PR)], src_a, semia)
    pltpu.async_copy(dst_hbm.at[pl.ds(r0, _PR)], dst_a, semia)
    for i in range(16):
        for j in range(_H // 16):
            zb[i, pl.ds(j * 16, 16)] = jnp.zeros((16,), jnp.float32)
    zcps = [
        pltpu.async_copy(zb, acc.at[pl.ds(s * _RPS + m * 16, 16)], semz)
        for m in range(_RPS // 16)
    ]
    for cp in zcps:
        cp.wait()
    plsc.subcore_barrier()

    bufs = (buf0, buf1)
    sems = (sem0, sem1)
    for ph in range(5):
        src_v, dst_v, semi = sets[ph % 2]
        rp = r0 + ph * _PR
        pltpu.make_async_copy(src_hbm.at[pl.ds(rp, _PR)], src_v, semi).wait()
        pltpu.make_async_copy(dst_hbm.at[pl.ds(rp, _PR)], dst_v, semi).wait()
        if ph + 1 < 5:
            nsrc, ndst, nsem = sets[(ph + 1) % 2]
            rn = r0 + (ph + 1) * _PR
            pltpu.async_copy(src_hbm.at[pl.ds(rn, _PR)], nsrc, nsem)
            pltpu.async_copy(dst_hbm.at[pl.ds(rn, _PR)], ndst, nsem)
        for b in range(2):
            pltpu.async_copy(y_hbm.at[src_v.at[b]], bufs[b], sems[b])

        def body(k, carry):
            j0 = 2 * k
            for b in range(2):
                j = j0 + b
                pltpu.make_async_copy(y_hbm.at[src_v.at[j]], bufs[b], sems[b]).wait()
                pltpu.sync_copy(bufs[b], acc.at[dst_v.at[j]], add=True)
                pltpu.async_copy(y_hbm.at[src_v.at[j + 2]], bufs[b], sems[b])
            return carry

        lax.fori_loop(0, (_PR - 2) // 2, body, 0)
        for b in range(2):
            j = _PR - 2 + b
            pltpu.make_async_copy(y_hbm.at[src_v.at[j]], bufs[b], sems[b]).wait()
            pltpu.sync_copy(bufs[b], acc.at[dst_v.at[j]], add=True)
    plsc.subcore_barrier()
    pltpu.sync_copy(
        acc.at[pl.ds(s * _RPS, _RPS)], out_hbm.at[c, pl.ds(s * _RPS, _RPS)]
    )


def _tc0_body(degp_ref, x_ref, ew_ref, eb_ref, w0_ref, dinv_ref, y_ref):
    deg = degp_ref[0, :, 0:1] + degp_ref[1, :, 0:1] + 1.0
    dinv = lax.rsqrt(deg)
    dinv_ref[...] = dinv
    h = jnp.dot(x_ref[...], ew_ref[...], preferred_element_type=jnp.float32)
    h = h + eb_ref[...]
    y_ref[...] = jnp.dot(h, w0_ref[...], preferred_element_type=jnp.float32) * dinv


_tc0 = pl.pallas_call(
    _tc0_body,
    grid=(_NP // _BR,),
    in_specs=[
        pl.BlockSpec((2, _BR, _DW), lambda i: (0, i, 0)),
        pl.BlockSpec((_BR, _H), lambda i: (i, 0)),
        pl.BlockSpec((_H, _H), lambda i: (0, 0)),
        pl.BlockSpec((1, _H), lambda i: (0, 0)),
        pl.BlockSpec((_H, _H), lambda i: (0, 0)),
    ],
    out_specs=[
        pl.BlockSpec((_BR, 1), lambda i: (i, 0)),
        pl.BlockSpec((_BR, _H), lambda i: (i, 0)),
    ],
    out_shape=[
        jax.ShapeDtypeStruct((_NP, 1), jnp.float32),
        jax.ShapeDtypeStruct((_NP, _H), jnp.float32),
    ],
)


def _tc_mid_body(p_ref, yp_ref, dinv_ref, b_ref, w_ref, y_ref):
    dinv = dinv_ref[...]
    h = dinv * (p_ref[0] + p_ref[1] + yp_ref[...]) + b_ref[...]
    y_ref[...] = jnp.dot(h, w_ref[...], preferred_element_type=jnp.float32) * dinv


_tc_mid = pl.pallas_call(
    _tc_mid_body,
    grid=(_NP // _BR,),
    in_specs=[
        pl.BlockSpec((2, _BR, _H), lambda i: (0, i, 0)),
        pl.BlockSpec((_BR, _H), lambda i: (i, 0)),
        pl.BlockSpec((_BR, 1), lambda i: (i, 0)),
        pl.BlockSpec((1, _H), lambda i: (0, 0)),
        pl.BlockSpec((_H, _H), lambda i: (0, 0)),
    ],
    out_specs=pl.BlockSpec((_BR, _H), lambda i: (i, 0)),
    out_shape=jax.ShapeDtypeStruct((_NP, _H), jnp.float32),
)


def _tc_fin_body(p_ref, yp_ref, dinv_ref, b_ref, w_ref, db_ref, o_ref):
    h = dinv_ref[...] * (p_ref[0] + p_ref[1] + yp_ref[...]) + b_ref[...]
    o_ref[...] = jnp.dot(h, w_ref[...], preferred_element_type=jnp.float32) + db_ref[...]


_tc_fin = pl.pallas_call(
    _tc_fin_body,
    grid=(_NP // _BR,),
    in_specs=[
        pl.BlockSpec((2, _BR, _H), lambda i: (0, i, 0)),
        pl.BlockSpec((_BR, _H), lambda i: (i, 0)),
        pl.BlockSpec((_BR, 1), lambda i: (i, 0)),
        pl.BlockSpec((1, _H), lambda i: (0, 0)),
        pl.BlockSpec((_H, _H), lambda i: (0, 0)),
        pl.BlockSpec((1, _H), lambda i: (0, 0)),
    ],
    out_specs=pl.BlockSpec((_BR, _H), lambda i: (i, 0)),
    out_shape=jax.ShapeDtypeStruct((_N, _H), jnp.float32),
)


def kernel(x, edge_index, emb_W, emb_b, W0, b0, W1, b1, W2, b2, dec_W, dec_b):
    src = edge_index[0]
    dst = edge_index[1]
    pad = _N + (jnp.arange(_EP - _E, dtype=edge_index.dtype) % (_NP - _N))
    src2 = jnp.concatenate([src, pad]).reshape(_ER, 128)
    dst2 = jnp.concatenate([dst, pad]).reshape(_ER, 128)
    x_p = jnp.concatenate([x, jnp.zeros((_NP - _N, x.shape[1]), x.dtype)])

    degp = _sc_deg(dst2)
    dinv, y = _tc0(degp, x_p, emb_W, emb_b.reshape(1, _H), W0)
    p = _sc_agg(y, src2, dst2)
    y = _tc_mid(p, y, dinv, b0.reshape(1, _H), W1)
    p = _sc_agg(y, src2, dst2)
    y = _tc_mid(p, y, dinv, b1.reshape(1, _H), W2)
    p = _sc_agg(y, src2, dst2)
    out = _tc_fin(p, y, dinv, b2.reshape(1, _H), dec_W, dec_b.reshape(1, _H))
    return out
